# Initial kernel scaffold; baseline (speedup 1.0000x reference)
import jax
import jax.numpy as jnp
from jax.experimental import pallas as pl


def kernel(x, edge_index, residual, W1, b1, W2, b2, gamma, beta):
    raise NotImplementedError("write your pallas kernel here")


# speedup vs baseline: 4.9929x; 4.9929x over previous
import functools

import jax
import jax.numpy as jnp
from jax import lax
from jax.experimental import pallas as pl
from jax.experimental.pallas import tpu as pltpu
from jax.experimental.pallas import tpu_sc as plsc

N_NODES = 10000
N_EDGES = 320000
HIDDEN = 128

NC = 2
NS = 16
NW = NC * NS

CHUNK = 128
CHUNKS_PER_TILE = -(-N_EDGES // (NW * CHUNK))
E_TILE = CHUNKS_PER_TILE * CHUNK
E_PAD = E_TILE * NW

ROWS_PER_TILE = 640
N_PAD = ROWS_PER_TILE * NS
JUNK_ROW = N_NODES


def _sc_body(src_hbm, dst_hbm, x_hbm, out_hbm, src_v, dst_v, rows_v, agg_sh, sem):
    c = lax.axis_index("c")
    s = lax.axis_index("s")
    wid = c * NS + s

    def zrow(r, carry):
        for k in range(HIDDEN // 16):
            rows_v[r, pl.ds(k * 16, 16)] = jnp.zeros((16,), jnp.float32)
        return carry

    lax.fori_loop(0, CHUNK, zrow, 0)

    def zcpy(i, carry):
        pltpu.sync_copy(rows_v, agg_sh.at[pl.ds(s * ROWS_PER_TILE + i * CHUNK, CHUNK)])
        return carry

    lax.fori_loop(0, ROWS_PER_TILE // CHUNK, zcpy, 0)

    pltpu.sync_copy(src_hbm.at[wid], src_v)
    pltpu.sync_copy(dst_hbm.at[wid], dst_v)

    plsc.subcore_barrier()

    def step(j, carry):
        pltpu.async_copy(x_hbm.at[src_v.at[j]], rows_v, sem).wait()
        pltpu.sync_copy(rows_v, agg_sh.at[dst_v.at[j]], add=True)
        return carry

    lax.fori_loop(0, CHUNKS_PER_TILE, step, 0)

    plsc.subcore_barrier()

    pltpu.sync_copy(
        agg_sh.at[pl.ds(s * ROWS_PER_TILE, ROWS_PER_TILE)],
        out_hbm.at[c, pl.ds(s * ROWS_PER_TILE, ROWS_PER_TILE)],
    )


def _sc_aggregate(src3, dst3, x):
    mesh = plsc.VectorSubcoreMesh(
        core_axis_name="c", subcore_axis_name="s", num_cores=NC, num_subcores=NS
    )
    return pl.kernel(
        _sc_body,
        out_type=jax.ShapeDtypeStruct((NC, N_PAD, HIDDEN), jnp.float32),
        mesh=mesh,
        scratch_types=[
            pltpu.VMEM((CHUNKS_PER_TILE, CHUNK), jnp.int32),
            pltpu.VMEM((CHUNKS_PER_TILE, CHUNK), jnp.int32),
            pltpu.VMEM((CHUNK, HIDDEN), jnp.float32),
            pltpu.VMEM_SHARED((N_PAD, HIDDEN), jnp.float32),
            pltpu.SemaphoreType.DMA,
        ],
    )(src3, dst3, x)


BLK = 1000
GRID = N_NODES // BLK


def _mlp_body(x_ref, agg_ref, w1_ref, b1_ref, w2_ref, b2_ref,
              h2_ref, sum_ref, ssq_ref):
    i = pl.program_id(0)
    h = x_ref[...] + agg_ref[0] + agg_ref[1]
    h1 = jnp.dot(h, w1_ref[...], preferred_element_type=jnp.float32) + b1_ref[...]
    h1 = jnp.maximum(h1, 0.0)
    h2 = jnp.dot(h1, w2_ref[...], preferred_element_type=jnp.float32) + b2_ref[...]
    h2_ref[...] = h2

    @pl.when(i == 0)
    def _():
        sum_ref[...] = jnp.zeros_like(sum_ref)
        ssq_ref[...] = jnp.zeros_like(ssq_ref)

    sum_ref[...] += jnp.sum(h2, axis=0, keepdims=True)
    ssq_ref[...] += jnp.sum(h2 * h2, axis=0, keepdims=True)


def _mlp_stats(x, agg2, W1, b1, W2, b2):
    return pl.pallas_call(
        _mlp_body,
        grid=(GRID,),
        in_specs=[
            pl.BlockSpec((BLK, HIDDEN), lambda i: (i, 0)),
            pl.BlockSpec((NC, BLK, HIDDEN), lambda i: (0, i, 0)),
            pl.BlockSpec((HIDDEN, HIDDEN), lambda i: (0, 0)),
            pl.BlockSpec((1, HIDDEN), lambda i: (0, 0)),
            pl.BlockSpec((HIDDEN, HIDDEN), lambda i: (0, 0)),
            pl.BlockSpec((1, HIDDEN), lambda i: (0, 0)),
        ],
        out_specs=[
            pl.BlockSpec((BLK, HIDDEN), lambda i: (i, 0)),
            pl.BlockSpec((1, HIDDEN), lambda i: (0, 0)),
            pl.BlockSpec((1, HIDDEN), lambda i: (0, 0)),
        ],
        out_shape=[
            jax.ShapeDtypeStruct((N_NODES, HIDDEN), jnp.float32),
            jax.ShapeDtypeStruct((1, HIDDEN), jnp.float32),
            jax.ShapeDtypeStruct((1, HIDDEN), jnp.float32),
        ],
    )(x, agg2, W1, b1.reshape(1, HIDDEN), W2, b2.reshape(1, HIDDEN))


def _bn_body(h2_ref, sum_ref, ssq_ref, gamma_ref, beta_ref, res_ref, out_ref):
    n = jnp.float32(N_NODES)
    mean = sum_ref[...] / n
    var = ssq_ref[...] / n - mean * mean
    rstd = lax.rsqrt(var + 1e-5)
    normed = (h2_ref[...] - mean) * rstd * gamma_ref[...] + beta_ref[...]
    out_ref[...] = jnp.maximum(normed, 0.0) + res_ref[...]


def _bn_residual(h2, ssum, ssq, gamma, beta, residual):
    return pl.pallas_call(
        _bn_body,
        grid=(GRID,),
        in_specs=[
            pl.BlockSpec((BLK, HIDDEN), lambda i: (i, 0)),
            pl.BlockSpec((1, HIDDEN), lambda i: (0, 0)),
            pl.BlockSpec((1, HIDDEN), lambda i: (0, 0)),
            pl.BlockSpec((1, HIDDEN), lambda i: (0, 0)),
            pl.BlockSpec((1, HIDDEN), lambda i: (0, 0)),
            pl.BlockSpec((BLK, HIDDEN), lambda i: (i, 0)),
        ],
        out_specs=pl.BlockSpec((BLK, HIDDEN), lambda i: (i, 0)),
        out_shape=jax.ShapeDtypeStruct((N_NODES, HIDDEN), jnp.float32),
    )(h2, ssum, ssq, gamma.reshape(1, HIDDEN), beta.reshape(1, HIDDEN), residual)


def kernel(x, edge_index, residual, W1, b1, W2, b2, gamma, beta):
    ei = edge_index.astype(jnp.int32)
    pad = E_PAD - N_EDGES
    src = jnp.concatenate([ei[0], jnp.zeros((pad,), jnp.int32)])
    dst = jnp.concatenate([ei[1], jnp.full((pad,), JUNK_ROW, jnp.int32)])
    src3 = src.reshape(NW, CHUNKS_PER_TILE, CHUNK)
    dst3 = dst.reshape(NW, CHUNKS_PER_TILE, CHUNK)

    agg2 = _sc_aggregate(src3, dst3, x)
    h2, ssum, ssq = _mlp_stats(x, agg2, W1, b1, W2, b2)
    return _bn_residual(h2, ssum, ssq, gamma, beta, residual)
